# baseline (speedup 1.0000x reference)
import functools

import jax
import jax.numpy as jnp
from jax import lax
from jax.experimental import pallas as pl
from jax.experimental.pallas import tpu as pltpu
from jax.experimental.pallas import tpu_sc as plsc

B = 16384
D = 32
NC = 2
NS = 16
NW = NC * NS
BPW = B // NW
CHUNK = 128
NCH = BPW // CHUNK

BQ = 2048


def _ceil_div(a, b):
    return -(-a // b)


@functools.cache
def _pack_fn(v_rows):
    nb = _ceil_div(v_rows, 4 * BQ)
    max_blk = v_rows // BQ - 1

    def imap(i, k):
        b = 4 * i + k
        return (0, b if k == 0 else jnp.minimum(b, max_blk))

    def body(t0, t1, t2, t3, tout):
        bf16 = jnp.bfloat16
        row = lax.broadcasted_iota(jnp.int32, (32, 128), 0)
        col = lax.broadcasted_iota(jnp.int32, (32, 128), 1)
        dn = (((0,), (0,)), ((), ()))

        def tr(ref, k):
            ek = (col == row + 32 * k).astype(bf16)
            return lax.dot_general(ref[...].astype(bf16), ek, dn,
                                   preferred_element_type=jnp.float32)

        s = tr(t0, 0) + tr(t1, 1) + tr(t2, 2) + tr(t3, 3)
        tout[...] = s.astype(bf16)

    return pl.pallas_call(
        body,
        grid=(nb,),
        in_specs=[
            pl.BlockSpec((32, BQ), lambda i, k=k: imap(i, k))
            for k in range(4)
        ],
        out_specs=pl.BlockSpec((BQ, 128), lambda i: (i, 0)),
        out_shape=jax.ShapeDtypeStruct((nb * BQ, 128), jnp.bfloat16),
        compiler_params=pltpu.CompilerParams(
            fuse_transposed_lhs_in_matmul=True,
        ),
    )


def _pack(table):
    tt = table.T
    return _pack_fn(table.shape[0])(tt, tt, tt, tt)


def _sc_gather_body(idx0, idx1, idx2, idx3, tab0, tab1, tab2, tab3,
                    out, qv, pk, obuf, sem, sem2):
    wid = lax.axis_index("s") * NC + lax.axis_index("c")
    base = wid * BPW
    idxs = (idx0, idx1, idx2, idx3)
    tabs = (tab0, tab1, tab2, tab3)
    for t in range(4):
        pltpu.sync_copy(idxs[t].at[wid], qv)
        handles = []
        for c in range(NCH):
            handles.append(pltpu.async_copy(
                tabs[t].at[qv.at[c]],
                pk.at[t].at[pl.ds(c * CHUNK, CHUNK)], sem))
        for h in handles:
            h.wait()

    oh = {}
    for c in range(NCH):
        def step(i, _, c=c):
            for t in range(4):
                obuf[c % 2, i, pl.ds(t * 32, 32)] = (
                    pk[t, c * CHUNK + i, pl.ds(0, 32)])
            return 0

        if c >= 2:
            oh[c - 2].wait()
        lax.fori_loop(0, CHUNK, step, 0)
        oh[c] = pltpu.async_copy(
            obuf.at[c % 2], out.at[pl.ds(base + c * CHUNK, CHUNK)], sem2)
    oh[NCH - 2].wait()
    oh[NCH - 1].wait()


@functools.cache
def _sc_gather_fn(vp0, vp1, vp2, vp3):
    mesh = plsc.VectorSubcoreMesh(
        core_axis_name="c", subcore_axis_name="s",
        num_cores=NC, num_subcores=NS)
    return pl.kernel(
        _sc_gather_body,
        out_type=jax.ShapeDtypeStruct((B, 128), jnp.bfloat16),
        mesh=mesh,
        scratch_types=[
            pltpu.VMEM((NCH, CHUNK), jnp.int32),
            pltpu.VMEM((4, BPW, D), jnp.bfloat16),
            pltpu.VMEM((2, CHUNK, 128), jnp.bfloat16),
            pltpu.SemaphoreType.DMA,
            pltpu.SemaphoreType.DMA,
        ],
        compiler_params=pltpu.CompilerParams(use_tc_tiling_on_sc=False),
    )


_BLK = 2048
_GRID = B // _BLK


def _dense_body(num_ref, vec_ref, pk_ref,
                wn1_ref, bn1_ref, wn2_ref, bn2_ref, wv_ref, bv_ref,
                wm1_ref, bm1_ref, wm2_ref, bm2_ref, out_ref):
    f32 = jnp.float32
    h = jnp.dot(num_ref[...], wn1_ref[...], preferred_element_type=f32)
    h = jnp.maximum(h + bn1_ref[...], 0.0)
    h = jnp.dot(h, wn2_ref[...], preferred_element_type=f32) + bn2_ref[...]
    v = jnp.dot(vec_ref[...], wv_ref[...], preferred_element_type=f32) + bv_ref[...]
    wm1 = wm1_ref[...]
    x = jnp.dot(pk_ref[...].astype(f32), wm1[0:128], preferred_element_type=f32)
    x = x + jnp.dot(h, wm1[128:224], preferred_element_type=f32)
    x = x + jnp.dot(v, wm1[224:256], preferred_element_type=f32)
    x = jnp.maximum(x + bm1_ref[...], 0.0)
    out_ref[...] = jnp.dot(x, wm2_ref[...], preferred_element_type=f32) + bm2_ref[...]


def _full(shape):
    return pl.BlockSpec(shape, lambda i: (0, 0))


_dense = pl.pallas_call(
    _dense_body,
    grid=(_GRID,),
    in_specs=[
        pl.BlockSpec((_BLK, 4), lambda i: (i, 0)),
        pl.BlockSpec((_BLK, 128), lambda i: (i, 0)),
        pl.BlockSpec((_BLK, 128), lambda i: (i, 0)),
        _full((4, 96)), _full((1, 96)),
        _full((96, 96)), _full((1, 96)),
        _full((128, 32)), _full((1, 32)),
        _full((256, 128)), _full((1, 128)),
        _full((128, 32)), _full((1, 32)),
    ],
    out_specs=pl.BlockSpec((_BLK, D), lambda i: (i, 0)),
    out_shape=jax.ShapeDtypeStruct((B, D), jnp.float32),
    compiler_params=pltpu.CompilerParams(
        dimension_semantics=("arbitrary",),
    ),
)


def kernel(item_id, cat_0, cat_1, cat_2, num_0, num_1, num_2, num_3, vec_0,
           item_table, cat_table_0, cat_table_1, cat_table_2,
           W_num1, b_num1, W_num2, b_num2, W_vec, b_vec,
           W_m1, b_m1, W_m2, b_m2):
    p0 = _pack(cat_table_0).reshape(-1, D)
    p1 = _pack(cat_table_1).reshape(-1, D)
    p2 = _pack(cat_table_2).reshape(-1, D)
    p3 = _pack(item_table).reshape(-1, D)

    def gidx(ix):
        r = ix.astype(jnp.int32)
        g = ((r >> 13) << 13) + ((r & 2047) << 2) + ((r >> 11) & 3)
        return g.reshape(NW, NCH, CHUNK)

    gather = _sc_gather_fn(p0.shape[0], p1.shape[0], p2.shape[0], p3.shape[0])
    pk = gather(gidx(cat_0), gidx(cat_1), gidx(cat_2), gidx(item_id),
                p0, p1, p2, p3)

    numerical_v = jnp.concatenate([num_0, num_1, num_2, num_3], axis=1)
    return _dense(
        numerical_v, vec_0, pk,
        W_num1, b_num1.reshape(1, -1), W_num2, b_num2.reshape(1, -1),
        W_vec, b_vec.reshape(1, -1),
        W_m1, b_m1.reshape(1, -1), W_m2, b_m2.reshape(1, -1))


# speedup vs baseline: 2.0212x; 2.0212x over previous
import functools

import jax
import jax.numpy as jnp
from jax import lax
from jax.experimental import pallas as pl
from jax.experimental.pallas import tpu as pltpu
from jax.experimental.pallas import tpu_sc as plsc

B = 16384
D = 32
NC = 2
NS = 16
NW = NC * NS
BPW = B // NW
CHUNK = 128
NCH = BPW // CHUNK

BQ = 2048


def _ceil_div(a, b):
    return -(-a // b)


@functools.cache
def _pack_fn(v_rows):
    nb = _ceil_div(v_rows, 4 * BQ)
    max_blk = v_rows // BQ - 1

    def imap(i, k):
        b = 4 * i + k
        return (0, b if k == 0 else jnp.minimum(b, max_blk))

    def body(t0, t1, t2, t3, tout):
        bf16 = jnp.bfloat16
        row = lax.broadcasted_iota(jnp.int32, (32, 128), 0)
        col = lax.broadcasted_iota(jnp.int32, (32, 128), 1)
        dn = (((0,), (0,)), ((), ()))

        def tr(ref, k):
            ek = (col == row + 32 * k).astype(bf16)
            return lax.dot_general(ref[...].astype(bf16), ek, dn,
                                   preferred_element_type=jnp.float32)

        tout[...] = tr(t0, 0) + tr(t1, 1) + tr(t2, 2) + tr(t3, 3)

    return pl.pallas_call(
        body,
        grid=(nb,),
        in_specs=[
            pl.BlockSpec((32, BQ), lambda i, k=k: imap(i, k))
            for k in range(4)
        ],
        out_specs=pl.BlockSpec((BQ, 128), lambda i: (i, 0)),
        out_shape=jax.ShapeDtypeStruct((nb * BQ, 128), jnp.float32),
        compiler_params=pltpu.CompilerParams(
            fuse_transposed_lhs_in_matmul=True,
        ),
    )


def _pack(table):
    tt = table.T
    return _pack_fn(table.shape[0])(tt, tt, tt, tt)


def _sc_gather_body(idx0, idx1, idx2, idx3, tab0, tab1, tab2, tab3,
                    out, qv, pk, obuf, sem, sem2):
    wid = lax.axis_index("s") * NC + lax.axis_index("c")
    base = wid * BPW
    idxs = (idx0, idx1, idx2, idx3)
    tabs = (tab0, tab1, tab2, tab3)
    for t in range(4):
        pltpu.sync_copy(idxs[t].at[wid], qv)
        handles = []
        for c in range(NCH):
            handles.append(pltpu.async_copy(
                tabs[t].at[qv.at[c]],
                pk.at[t].at[pl.ds(c * CHUNK, CHUNK)], sem))
        for h in handles:
            h.wait()

    oh = {}
    for c in range(NCH):
        def step(i, _, c=c):
            for t in range(4):
                for half in range(2):
                    obuf[c % 2, i, pl.ds(t * 32 + half * 16, 16)] = (
                        pk[t, c * CHUNK + i, pl.ds(half * 16, 16)])
            return 0

        if c >= 2:
            oh[c - 2].wait()
        lax.fori_loop(0, CHUNK, step, 0)
        oh[c] = pltpu.async_copy(
            obuf.at[c % 2], out.at[pl.ds(base + c * CHUNK, CHUNK)], sem2)
    oh[NCH - 2].wait()
    oh[NCH - 1].wait()


@functools.cache
def _sc_gather_fn(vp0, vp1, vp2, vp3):
    mesh = plsc.VectorSubcoreMesh(
        core_axis_name="c", subcore_axis_name="s",
        num_cores=NC, num_subcores=NS)
    return pl.kernel(
        _sc_gather_body,
        out_type=jax.ShapeDtypeStruct((B, 128), jnp.float32),
        mesh=mesh,
        scratch_types=[
            pltpu.VMEM((NCH, CHUNK), jnp.int32),
            pltpu.VMEM((4, BPW, D), jnp.float32),
            pltpu.VMEM((2, CHUNK, 128), jnp.float32),
            pltpu.SemaphoreType.DMA,
            pltpu.SemaphoreType.DMA,
        ],
        compiler_params=pltpu.CompilerParams(use_tc_tiling_on_sc=False),
    )


_BLK = 2048
_GRID = B // _BLK


def _dense_body(num_ref, vec_ref, pk_ref,
                wn1_ref, bn1_ref, wn2_ref, bn2_ref, wv_ref, bv_ref,
                wm1_ref, bm1_ref, wm2_ref, bm2_ref, out_ref):
    f32 = jnp.float32
    h = jnp.dot(num_ref[...], wn1_ref[...], preferred_element_type=f32)
    h = jnp.maximum(h + bn1_ref[...], 0.0)
    h = jnp.dot(h, wn2_ref[...], preferred_element_type=f32) + bn2_ref[...]
    v = jnp.dot(vec_ref[...], wv_ref[...], preferred_element_type=f32) + bv_ref[...]
    wm1 = wm1_ref[...]
    x = jnp.dot(pk_ref[...], wm1[0:128], preferred_element_type=f32)
    x = x + jnp.dot(h, wm1[128:224], preferred_element_type=f32)
    x = x + jnp.dot(v, wm1[224:256], preferred_element_type=f32)
    x = jnp.maximum(x + bm1_ref[...], 0.0)
    out_ref[...] = jnp.dot(x, wm2_ref[...], preferred_element_type=f32) + bm2_ref[...]


def _full(shape):
    return pl.BlockSpec(shape, lambda i: (0, 0))


_dense = pl.pallas_call(
    _dense_body,
    grid=(_GRID,),
    in_specs=[
        pl.BlockSpec((_BLK, 4), lambda i: (i, 0)),
        pl.BlockSpec((_BLK, 128), lambda i: (i, 0)),
        pl.BlockSpec((_BLK, 128), lambda i: (i, 0)),
        _full((4, 96)), _full((1, 96)),
        _full((96, 96)), _full((1, 96)),
        _full((128, 32)), _full((1, 32)),
        _full((256, 128)), _full((1, 128)),
        _full((128, 32)), _full((1, 32)),
    ],
    out_specs=pl.BlockSpec((_BLK, D), lambda i: (i, 0)),
    out_shape=jax.ShapeDtypeStruct((B, D), jnp.float32),
    compiler_params=pltpu.CompilerParams(
        dimension_semantics=("arbitrary",),
    ),
)


def kernel(item_id, cat_0, cat_1, cat_2, num_0, num_1, num_2, num_3, vec_0,
           item_table, cat_table_0, cat_table_1, cat_table_2,
           W_num1, b_num1, W_num2, b_num2, W_vec, b_vec,
           W_m1, b_m1, W_m2, b_m2):
    p0 = _pack(cat_table_0).reshape(-1, D)
    p1 = _pack(cat_table_1).reshape(-1, D)
    p2 = _pack(cat_table_2).reshape(-1, D)
    p3 = _pack(item_table).reshape(-1, D)

    def gidx(ix):
        r = ix.astype(jnp.int32)
        g = ((r >> 13) << 13) + ((r & 2047) << 2) + ((r >> 11) & 3)
        return g.reshape(NW, NCH, CHUNK)

    gather = _sc_gather_fn(p0.shape[0], p1.shape[0], p2.shape[0], p3.shape[0])
    pk = gather(gidx(cat_0), gidx(cat_1), gidx(cat_2), gidx(item_id),
                p0, p1, p2, p3)

    numerical_v = jnp.concatenate([num_0, num_1, num_2, num_3], axis=1)
    return _dense(
        numerical_v, vec_0, pk,
        W_num1, b_num1.reshape(1, -1), W_num2, b_num2.reshape(1, -1),
        W_vec, b_vec.reshape(1, -1),
        W_m1, b_m1.reshape(1, -1), W_m2, b_m2.reshape(1, -1))


# speedup vs baseline: 2.5801x; 1.2765x over previous
import functools

import jax
import jax.numpy as jnp
from jax import lax
from jax.experimental import pallas as pl
from jax.experimental.pallas import tpu as pltpu
from jax.experimental.pallas import tpu_sc as plsc

B = 16384
D = 32
NC = 2
NS = 16
NW = NC * NS
BPW = B // NW
CHUNK = 128
NCH = BPW // CHUNK

BQ = 4096


def _ceil_div(a, b):
    return -(-a // b)


@functools.cache
def _pack_fn(v_rows):
    nb = _ceil_div(v_rows, 4 * BQ)
    max_blk = v_rows // BQ - 1

    def imap(i, k):
        b = 4 * i + k
        return (0, b if k == 0 else jnp.minimum(b, max_blk))

    def body(t0, t1, t2, t3, tout):
        bf16 = jnp.bfloat16
        row = lax.broadcasted_iota(jnp.int32, (32, 128), 0)
        col = lax.broadcasted_iota(jnp.int32, (32, 128), 1)
        dn = (((0,), (0,)), ((), ()))

        def tr(ref, k):
            ek = (col == row + 32 * k).astype(bf16)
            return lax.dot_general(ref[...].astype(bf16), ek, dn,
                                   preferred_element_type=jnp.float32)

        tout[...] = tr(t0, 0) + tr(t1, 1) + tr(t2, 2) + tr(t3, 3)

    return pl.pallas_call(
        body,
        grid=(nb,),
        in_specs=[
            pl.BlockSpec((32, BQ), lambda i, k=k: imap(i, k))
            for k in range(4)
        ],
        out_specs=pl.BlockSpec((BQ, 128), lambda i: (i, 0)),
        out_shape=jax.ShapeDtypeStruct((nb * BQ, 128), jnp.float32),
        compiler_params=pltpu.CompilerParams(
            fuse_transposed_lhs_in_matmul=True,
        ),
    )


def _pack(table):
    tt = table.T
    return _pack_fn(table.shape[0])(tt, tt, tt, tt)


@functools.cache
def _pack3_fn(v_rows):
    nb = _ceil_div(v_rows, 4 * BQ)
    max_blk = v_rows // BQ - 1

    def imap(i, k):
        b = 4 * i + k
        return (0, b if k == 0 else jnp.minimum(b, max_blk))

    def body(*refs):
        ins, outs = refs[:12], refs[12:]
        bf16 = jnp.bfloat16
        row = lax.broadcasted_iota(jnp.int32, (32, 128), 0)
        col = lax.broadcasted_iota(jnp.int32, (32, 128), 1)
        dn = (((0,), (0,)), ((), ()))

        def tr(ref, k):
            ek = (col == row + 32 * k).astype(bf16)
            return lax.dot_general(ref[...].astype(bf16), ek, dn,
                                   preferred_element_type=jnp.float32)

        for j in range(3):
            outs[j][...] = sum(tr(ins[4 * j + k], k) for k in range(1, 4)) \
                + tr(ins[4 * j], 0)

    return pl.pallas_call(
        body,
        grid=(nb,),
        in_specs=[
            pl.BlockSpec((32, BQ), lambda i, k=k: imap(i, k))
            for _ in range(3) for k in range(4)
        ],
        out_specs=[pl.BlockSpec((BQ, 128), lambda i: (i, 0))] * 3,
        out_shape=[jax.ShapeDtypeStruct((nb * BQ, 128), jnp.float32)] * 3,
        compiler_params=pltpu.CompilerParams(
            fuse_transposed_lhs_in_matmul=True,
        ),
    )


def _pack3(t0, t1, t2):
    tts = [t.T for t in (t0, t1, t2)]
    args = [tt for tt in tts for _ in range(4)]
    return _pack3_fn(t0.shape[0])(*args)


def _sc_gather_body(idx0, idx1, idx2, idx3, tab0, tab1, tab2, tab3,
                    out, qv, pk, obuf, semi, sem0, sem1, sem2, sem3, semo):
    wid = lax.axis_index("s") * NC + lax.axis_index("c")
    base = wid * BPW
    idxs = (idx0, idx1, idx2, idx3)
    tabs = (tab0, tab1, tab2, tab3)
    sems = (sem0, sem1, sem2, sem3)
    ih = [pltpu.async_copy(idxs[t].at[wid], qv.at[t], semi) for t in range(4)]
    for h in ih:
        h.wait()
    gh = {c: [pltpu.async_copy(tabs[t].at[qv.at[t, c]],
                               pk.at[t].at[pl.ds(c * CHUNK, CHUNK)], sems[c])
              for t in range(4)]
          for c in range(NCH)}

    oh = {}
    for c in range(NCH):
        def step(i4, _, c=c):
            for u in range(4):
                i = i4 * 4 + u
                for t in range(4):
                    for half in range(2):
                        obuf[c % 2, i, pl.ds(t * 32 + half * 16, 16)] = (
                            pk[t, c * CHUNK + i, pl.ds(half * 16, 16)])
            return 0

        for h in gh[c]:
            h.wait()
        if c >= 2:
            oh[c - 2].wait()
        lax.fori_loop(0, CHUNK // 4, step, 0)
        oh[c] = pltpu.async_copy(
            obuf.at[c % 2], out.at[pl.ds(base + c * CHUNK, CHUNK)], semo)
    oh[NCH - 2].wait()
    oh[NCH - 1].wait()


@functools.cache
def _sc_gather_fn(vp0, vp1, vp2, vp3):
    mesh = plsc.VectorSubcoreMesh(
        core_axis_name="c", subcore_axis_name="s",
        num_cores=NC, num_subcores=NS)
    return pl.kernel(
        _sc_gather_body,
        out_type=jax.ShapeDtypeStruct((B, 128), jnp.float32),
        mesh=mesh,
        scratch_types=[
            pltpu.VMEM((4, NCH, CHUNK), jnp.int32),
            pltpu.VMEM((4, BPW, D), jnp.float32),
            pltpu.VMEM((2, CHUNK, 128), jnp.float32),
            pltpu.SemaphoreType.DMA,
            pltpu.SemaphoreType.DMA,
            pltpu.SemaphoreType.DMA,
            pltpu.SemaphoreType.DMA,
            pltpu.SemaphoreType.DMA,
            pltpu.SemaphoreType.DMA,
        ],
        compiler_params=pltpu.CompilerParams(use_tc_tiling_on_sc=False),
    )


_BLK = 2048
_GRID = B // _BLK


def _dense_body(num_ref, vec_ref, pk_ref,
                wn1_ref, bn1_ref, wn2_ref, bn2_ref, wv_ref, bv_ref,
                wm1_ref, bm1_ref, wm2_ref, bm2_ref, out_ref):
    f32 = jnp.float32
    h = jnp.dot(num_ref[...], wn1_ref[...], preferred_element_type=f32)
    h = jnp.maximum(h + bn1_ref[...], 0.0)
    h = jnp.dot(h, wn2_ref[...], preferred_element_type=f32) + bn2_ref[...]
    v = jnp.dot(vec_ref[...], wv_ref[...], preferred_element_type=f32) + bv_ref[...]
    wm1 = wm1_ref[...]
    x = jnp.dot(pk_ref[...], wm1[0:128], preferred_element_type=f32)
    x = x + jnp.dot(h, wm1[128:224], preferred_element_type=f32)
    x = x + jnp.dot(v, wm1[224:256], preferred_element_type=f32)
    x = jnp.maximum(x + bm1_ref[...], 0.0)
    out_ref[...] = jnp.dot(x, wm2_ref[...], preferred_element_type=f32) + bm2_ref[...]


def _full(shape):
    return pl.BlockSpec(shape, lambda i: (0, 0))


_dense = pl.pallas_call(
    _dense_body,
    grid=(_GRID,),
    in_specs=[
        pl.BlockSpec((_BLK, 4), lambda i: (i, 0)),
        pl.BlockSpec((_BLK, 128), lambda i: (i, 0)),
        pl.BlockSpec((_BLK, 128), lambda i: (i, 0)),
        _full((4, 96)), _full((1, 96)),
        _full((96, 96)), _full((1, 96)),
        _full((128, 32)), _full((1, 32)),
        _full((256, 128)), _full((1, 128)),
        _full((128, 32)), _full((1, 32)),
    ],
    out_specs=pl.BlockSpec((_BLK, D), lambda i: (i, 0)),
    out_shape=jax.ShapeDtypeStruct((B, D), jnp.float32),
    compiler_params=pltpu.CompilerParams(
        dimension_semantics=("arbitrary",),
    ),
)


def kernel(item_id, cat_0, cat_1, cat_2, num_0, num_1, num_2, num_3, vec_0,
           item_table, cat_table_0, cat_table_1, cat_table_2,
           W_num1, b_num1, W_num2, b_num2, W_vec, b_vec,
           W_m1, b_m1, W_m2, b_m2):
    c0p, c1p, c2p = _pack3(cat_table_0, cat_table_1, cat_table_2)
    p0, p1, p2 = (c0p.reshape(-1, D), c1p.reshape(-1, D), c2p.reshape(-1, D))
    p3 = _pack(item_table).reshape(-1, D)

    def gidx(ix):
        r = ix.astype(jnp.int32)
        g = ((r >> 14) << 14) + ((r & 4095) << 2) + ((r >> 12) & 3)
        return g.reshape(NW, NCH, CHUNK)

    gather = _sc_gather_fn(p0.shape[0], p1.shape[0], p2.shape[0], p3.shape[0])
    pk = gather(gidx(cat_0), gidx(cat_1), gidx(cat_2), gidx(item_id),
                p0, p1, p2, p3)

    numerical_v = jnp.concatenate([num_0, num_1, num_2, num_3], axis=1)
    return _dense(
        numerical_v, vec_0, pk,
        W_num1, b_num1.reshape(1, -1), W_num2, b_num2.reshape(1, -1),
        W_vec, b_vec.reshape(1, -1),
        W_m1, b_m1.reshape(1, -1), W_m2, b_m2.reshape(1, -1))


# speedup vs baseline: 2.7707x; 1.0739x over previous
import functools

import jax
import jax.numpy as jnp
from jax import lax
from jax.experimental import pallas as pl
from jax.experimental.pallas import tpu as pltpu
from jax.experimental.pallas import tpu_sc as plsc

B = 16384
D = 32
NC = 2
NS = 16
NW = NC * NS
BPW = B // NW
CHUNK = 128
NCH = BPW // CHUNK

BQ = 4096


def _ceil_div(a, b):
    return -(-a // b)


@functools.cache
def _pack_fn(v_rows):
    nb = _ceil_div(v_rows, 4 * BQ)
    max_blk = v_rows // BQ - 1

    def imap(i, k):
        b = 4 * i + k
        return (0, b if k == 0 else jnp.minimum(b, max_blk))

    def body(t0, t1, t2, t3, tout):
        bf16 = jnp.bfloat16
        row = lax.broadcasted_iota(jnp.int32, (32, 128), 0)
        col = lax.broadcasted_iota(jnp.int32, (32, 128), 1)
        dn = (((0,), (0,)), ((), ()))

        def tr(ref, k):
            ek = (col == row + 32 * k).astype(bf16)
            return lax.dot_general(ref[...].astype(bf16), ek, dn,
                                   preferred_element_type=jnp.float32)

        tout[...] = tr(t0, 0) + tr(t1, 1) + tr(t2, 2) + tr(t3, 3)

    return pl.pallas_call(
        body,
        grid=(nb,),
        in_specs=[
            pl.BlockSpec((32, BQ), lambda i, k=k: imap(i, k))
            for k in range(4)
        ],
        out_specs=pl.BlockSpec((BQ, 128), lambda i: (i, 0)),
        out_shape=jax.ShapeDtypeStruct((nb * BQ, 128), jnp.float32),
        compiler_params=pltpu.CompilerParams(
            fuse_transposed_lhs_in_matmul=True,
        ),
    )


def _pack(table):
    tt = table.T
    return _pack_fn(table.shape[0])(tt, tt, tt, tt)


@functools.cache
def _pack3_fn(v_rows):
    nb = _ceil_div(v_rows, 4 * BQ)
    max_blk = v_rows // BQ - 1

    def imap(i, k):
        b = 4 * i + k
        return (0, b if k == 0 else jnp.minimum(b, max_blk))

    def body(*refs):
        ins, outs = refs[:12], refs[12:]
        bf16 = jnp.bfloat16
        row = lax.broadcasted_iota(jnp.int32, (32, 128), 0)
        col = lax.broadcasted_iota(jnp.int32, (32, 128), 1)
        dn = (((0,), (0,)), ((), ()))

        def tr(ref, k):
            ek = (col == row + 32 * k).astype(bf16)
            return lax.dot_general(ref[...].astype(bf16), ek, dn,
                                   preferred_element_type=jnp.float32)

        for j in range(3):
            outs[j][...] = sum(tr(ins[4 * j + k], k) for k in range(1, 4)) \
                + tr(ins[4 * j], 0)

    return pl.pallas_call(
        body,
        grid=(nb,),
        in_specs=[
            pl.BlockSpec((32, BQ), lambda i, k=k: imap(i, k))
            for _ in range(3) for k in range(4)
        ],
        out_specs=[pl.BlockSpec((BQ, 128), lambda i: (i, 0))] * 3,
        out_shape=[jax.ShapeDtypeStruct((nb * BQ, 128), jnp.float32)] * 3,
        compiler_params=pltpu.CompilerParams(
            fuse_transposed_lhs_in_matmul=True,
        ),
    )


def _pack3(t0, t1, t2):
    tts = [t.T for t in (t0, t1, t2)]
    args = [tt for tt in tts for _ in range(4)]
    return _pack3_fn(t0.shape[0])(*args)


def _make_sc_gather_body(ntab):
    def body(*refs):
        idxs = refs[:ntab]
        tabs = refs[ntab:2 * ntab]
        out = refs[2 * ntab]
        qv, pk, obuf = refs[2 * ntab + 1:2 * ntab + 4]
        semi, sem0, sem1, sem2, sem3, semo = refs[2 * ntab + 4:]
        sems = (sem0, sem1, sem2, sem3)
        wid = lax.axis_index("s") * NC + lax.axis_index("c")
        base = wid * BPW
        ih = [pltpu.async_copy(idxs[t].at[wid], qv.at[t], semi)
              for t in range(ntab)]
        for h in ih:
            h.wait()
        gh = {c: [pltpu.async_copy(
                      tabs[t].at[qv.at[t, pl.ds(c * CHUNK, CHUNK)]],
                      pk.at[t].at[pl.ds(c * CHUNK, CHUNK)], sems[c])
                  for t in range(ntab)]
              for c in range(NCH)}

        oh = {}
        for c in range(NCH):
            def step(i4, _, c=c):
                for u in range(4):
                    i = i4 * 4 + u
                    for t in range(ntab):
                        for half in range(2):
                            obuf[c % 2, i, pl.ds(t * 32 + half * 16, 16)] = (
                                pk[t, c * CHUNK + i, pl.ds(half * 16, 16)])
                return 0

            for h in gh[c]:
                h.wait()
            if c >= 2:
                oh[c - 2].wait()
            lax.fori_loop(0, CHUNK // 4, step, 0)
            oh[c] = pltpu.async_copy(
                obuf.at[c % 2], out.at[pl.ds(base + c * CHUNK, CHUNK)], semo)
        oh[NCH - 2].wait()
        oh[NCH - 1].wait()

    return body


@functools.cache
def _sc_gather_fn(ntab):
    mesh = plsc.VectorSubcoreMesh(
        core_axis_name="c", subcore_axis_name="s",
        num_cores=NC, num_subcores=NS)
    return pl.kernel(
        _make_sc_gather_body(ntab),
        out_type=jax.ShapeDtypeStruct((B, 128), jnp.float32),
        mesh=mesh,
        scratch_types=[
            pltpu.VMEM((ntab, NCH * CHUNK), jnp.int32),
            pltpu.VMEM((ntab, BPW, D), jnp.float32),
            pltpu.VMEM((2, CHUNK, 128), jnp.float32),
            pltpu.SemaphoreType.DMA,
            pltpu.SemaphoreType.DMA,
            pltpu.SemaphoreType.DMA,
            pltpu.SemaphoreType.DMA,
            pltpu.SemaphoreType.DMA,
            pltpu.SemaphoreType.DMA,
        ],
        compiler_params=pltpu.CompilerParams(use_tc_tiling_on_sc=False),
    )


_BLK = 2048
_GRID = B // _BLK


def _dense_body(num_ref, vec_ref, pkc_ref, pki_ref,
                wn1_ref, bn1_ref, wn2_ref, bn2_ref, wv_ref, bv_ref,
                wm1_ref, bm1_ref, wm2_ref, bm2_ref, out_ref):
    f32 = jnp.float32
    h = jnp.dot(num_ref[...], wn1_ref[...], preferred_element_type=f32)
    h = jnp.maximum(h + bn1_ref[...], 0.0)
    h = jnp.dot(h, wn2_ref[...], preferred_element_type=f32) + bn2_ref[...]
    v = jnp.dot(vec_ref[...], wv_ref[...], preferred_element_type=f32) + bv_ref[...]
    wm1 = wm1_ref[...]
    x = jnp.dot(pkc_ref[...][:, 0:96], wm1[0:96], preferred_element_type=f32)
    x = x + jnp.dot(pki_ref[...][:, 0:32], wm1[96:128],
                    preferred_element_type=f32)
    x = x + jnp.dot(h, wm1[128:224], preferred_element_type=f32)
    x = x + jnp.dot(v, wm1[224:256], preferred_element_type=f32)
    x = jnp.maximum(x + bm1_ref[...], 0.0)
    y = jnp.dot(x, wm2_ref[...], preferred_element_type=f32) + bm2_ref[...]
    out_ref[...] = y.T


def _full(shape):
    return pl.BlockSpec(shape, lambda i: (0, 0))


_dense = pl.pallas_call(
    _dense_body,
    grid=(_GRID,),
    in_specs=[
        pl.BlockSpec((_BLK, 4), lambda i: (i, 0)),
        pl.BlockSpec((_BLK, 128), lambda i: (i, 0)),
        pl.BlockSpec((_BLK, 128), lambda i: (i, 0)),
        pl.BlockSpec((_BLK, 128), lambda i: (i, 0)),
        _full((4, 96)), _full((1, 96)),
        _full((96, 96)), _full((1, 96)),
        _full((128, 32)), _full((1, 32)),
        _full((256, 128)), _full((1, 128)),
        _full((128, 32)), _full((1, 32)),
    ],
    out_specs=pl.BlockSpec((D, _BLK), lambda i: (0, i)),
    out_shape=jax.ShapeDtypeStruct((D, B), jnp.float32),
    compiler_params=pltpu.CompilerParams(
        dimension_semantics=("arbitrary",),
    ),
)


def kernel(item_id, cat_0, cat_1, cat_2, num_0, num_1, num_2, num_3, vec_0,
           item_table, cat_table_0, cat_table_1, cat_table_2,
           W_num1, b_num1, W_num2, b_num2, W_vec, b_vec,
           W_m1, b_m1, W_m2, b_m2):
    def gidx(ix):
        r = ix.astype(jnp.int32)
        g = ((r >> 14) << 14) + ((r & 4095) << 2) + ((r >> 12) & 3)
        return g.reshape(NW, NCH * CHUNK)

    c0p, c1p, c2p = _pack3(cat_table_0, cat_table_1, cat_table_2)
    pkc = _sc_gather_fn(3)(gidx(cat_0), gidx(cat_1), gidx(cat_2),
                           c0p.reshape(-1, D), c1p.reshape(-1, D),
                           c2p.reshape(-1, D))
    p3 = _pack(item_table).reshape(-1, D)
    pki = _sc_gather_fn(1)(gidx(item_id), p3)

    numerical_v = jnp.concatenate([num_0, num_1, num_2, num_3], axis=1)
    outT = _dense(
        numerical_v, vec_0, pkc, pki,
        W_num1, b_num1.reshape(1, -1), W_num2, b_num2.reshape(1, -1),
        W_vec, b_vec.reshape(1, -1),
        W_m1, b_m1.reshape(1, -1), W_m2, b_m2.reshape(1, -1))
    return outT.T
